# baseline (speedup 1.0000x reference)
import functools

import jax
import jax.numpy as jnp
import numpy as np
from jax.experimental import pallas as pl
from jax.experimental.pallas import tpu as pltpu
from jax.experimental.pallas import tpu_sc as plsc

NC = 64
T = 64
D = 512
H = 512
G3 = 3 * H
TT = 8


def _dot_t(a, b):
    return jax.lax.dot_general(a, b, (((1,), (1,)), ((), ())),
                               preferred_element_type=jnp.float32)


def _gru_gates(gi, gh, h):
    r = jax.nn.sigmoid(gi[:, :H] + gh[:, :H])
    z = jax.nn.sigmoid(gi[:, H:2 * H] + gh[:, H:2 * H])
    n = jnp.tanh(gi[:, 2 * H:] + r * gh[:, 2 * H:])
    return (1.0 - z) * n + z * h


def _fused_body(len_ref, img_ref, wihl_ref, bihl_ref, whhl_ref, bhhl_ref,
                wih_ref, bih_ref, whh_ref, bhh_ref,
                wst_ref, bst_ref, wct_ref, bct_ref,
                wa1_ref, ba1_ref, w2_ref, b2_ref,
                o_ref, h_ref):
    i = pl.program_id(0)

    @pl.when(i == 0)
    def _init():
        h_ref[...] = jnp.zeros_like(h_ref)

    xt = jnp.swapaxes(img_ref[...], 0, 1).reshape(TT * NC, D)
    lens = len_ref[...]
    gi_all = _dot_t(xt, wihl_ref[...]) + bihl_ref[...]
    whhl = whhl_ref[...]
    bhhl = bhhl_ref[...]
    h = h_ref[...]
    for tau in range(TT):
        t = i * TT + tau
        gi = gi_all[tau * NC:(tau + 1) * NC, :]
        gh = _dot_t(h, whhl) + bhhl
        h_new = _gru_gates(gi, gh, h)
        h = jnp.where(t < lens, h_new, h)
    h_ref[...] = h

    @pl.when(i == T // TT - 1)
    def _head():
        cr = h_ref[...]
        gih = _dot_t(cr, wih_ref[...]) + bih_ref[...]
        whh = whh_ref[...]
        bhh = bhh_ref[...]
        row_ids = jax.lax.broadcasted_iota(jnp.int32, (1, NC), 1)

        def step(k, hh):
            onehot = (row_ids == k).astype(jnp.float32)
            g = jnp.dot(onehot, gih, preferred_element_type=jnp.float32)
            gh2 = _dot_t(hh, whh) + bhh
            return _gru_gates(g, gh2, hh)

        h_hi = jax.lax.fori_loop(0, NC, step, jnp.zeros((1, H), jnp.float32))

        state = jax.nn.relu(_dot_t(h_hi, wst_ref[...]) + bst_ref[...])
        c1024 = jax.nn.relu(_dot_t(cr, wct_ref[...]) + bct_ref[...])
        wa1 = wa1_ref[...]
        s = _dot_t(state, wa1[:, :1024]) + ba1_ref[...]
        P = _dot_t(c1024, wa1[:, 1024:])
        A = P + s
        w2row = w2_ref[...].reshape(1, 1, 1024)

        CH = 8
        chunks = []
        for c in range(NC // CH):
            pc = P[c * CH:(c + 1) * CH, :]
            zq = jnp.maximum(A[:, None, :] + pc[None, :, :], 0.0)
            chunks.append(jnp.sum(zq * w2row, axis=2))
        tab = jnp.concatenate(chunks, axis=1) + b2_ref[...]

        rr = jax.lax.broadcasted_iota(jnp.int32, (NC, NC), 0)
        cc = jax.lax.broadcasted_iota(jnp.int32, (NC, NC), 1)
        valid = rr > cc
        tabm = jnp.where(valid, tab, jnp.float32(-1e30))
        m = jnp.max(tabm)
        e = jnp.where(valid, jnp.exp(tabm - m), 0.0)
        o_ref[...] = e / jnp.sum(e)


@jax.jit
def kernel(images, lengths, Wih_low, Whh_low, bih_low, bhh_low,
           Wih_high, Whh_high, bih_high, bhh_high,
           W_state, b_state, W_cluster, b_cluster,
           W_a1, b_a1, W_a2, b_a2):
    f32 = jnp.float32
    len2 = lengths.astype(jnp.int32).reshape(NC, 1)
    c = lambda shape: pl.BlockSpec(shape, lambda i: tuple(0 for _ in shape))
    probs = pl.pallas_call(
        _fused_body,
        grid=(T // TT,),
        in_specs=[
            c((NC, 1)),
            pl.BlockSpec((NC, TT, D), lambda i: (0, i, 0)),
            c((G3, D)),
            c((1, G3)),
            c((G3, H)),
            c((1, G3)),
            c((G3, H)),
            c((1, G3)),
            c((G3, H)),
            c((1, G3)),
            c((1024, H)),
            c((1, 1024)),
            c((1024, H)),
            c((1, 1024)),
            c((1024, 2048)),
            c((1, 1024)),
            c((1, 1024)),
            c((1, 1)),
        ],
        out_specs=pl.BlockSpec((NC, NC), lambda i: (0, 0)),
        out_shape=jax.ShapeDtypeStruct((NC, NC), f32),
        scratch_shapes=[pltpu.VMEM((NC, H), f32)],
    )(len2, images, Wih_low, bih_low.reshape(1, G3),
      Whh_low, bhh_low.reshape(1, G3),
      Wih_high, bih_high.reshape(1, G3),
      Whh_high, bhh_high.reshape(1, G3),
      W_state, b_state.reshape(1, 1024),
      W_cluster, b_cluster.reshape(1, 1024),
      W_a1, b_a1.reshape(1, 1024),
      W_a2, b_a2.reshape(1, 1))

    row_idx, col_idx = np.tril_indices(NC, k=-1)
    flat_idx = row_idx * NC + col_idx
    idx_pad = np.zeros((2048,), np.int32)
    idx_pad[:2016] = flat_idx
    q_flat = _sc_tril_gather(probs.reshape(NC * NC), jnp.asarray(idx_pad))
    return q_flat[:2016][:, None]


_SC_INFO = plsc.get_sparse_core_info()
_NWORK = _SC_INFO.num_cores * _SC_INFO.num_subcores
_CHUNK = 2048 // _NWORK


@jax.jit
@functools.partial(
    pl.kernel,
    mesh=plsc.VectorSubcoreMesh(core_axis_name="c", subcore_axis_name="s"),
    out_type=jax.ShapeDtypeStruct((2048,), jnp.float32),
    scratch_types=[
        pltpu.VMEM((_CHUNK,), jnp.int32),
        pltpu.VMEM((_CHUNK,), jnp.float32),
        pltpu.SemaphoreType.DMA,
    ],
)
def _sc_tril_gather(table_hbm, idx_hbm, out_hbm, idx_v, vals_v, sem):
    wid = jax.lax.axis_index("s") * _SC_INFO.num_cores + jax.lax.axis_index("c")
    base = wid * _CHUNK
    pltpu.sync_copy(idx_hbm.at[pl.ds(base, _CHUNK)], idx_v)
    pltpu.async_copy(table_hbm.at[idx_v], vals_v, sem).wait()
    pltpu.sync_copy(vals_v, out_hbm.at[pl.ds(base, _CHUNK)])


# speedup vs baseline: 1.0540x; 1.0540x over previous
import functools

import jax
import jax.numpy as jnp
import numpy as np
from jax.experimental import pallas as pl
from jax.experimental.pallas import tpu as pltpu
from jax.experimental.pallas import tpu_sc as plsc

NC = 64
T = 64
D = 512
H = 512
G3 = 3 * H
TT = 8


def _dot_t(a, b):
    return jax.lax.dot_general(a, b, (((1,), (1,)), ((), ())),
                               preferred_element_type=jnp.float32)


def _gru_gates(gi, gh, h):
    r = jax.nn.sigmoid(gi[:, :H] + gh[:, :H])
    z = jax.nn.sigmoid(gi[:, H:2 * H] + gh[:, H:2 * H])
    n = jnp.tanh(gi[:, 2 * H:] + r * gh[:, 2 * H:])
    return (1.0 - z) * n + z * h


def _fused_body(len_ref, img_ref, wihl_ref, bihl_ref, whhl_ref, bhhl_ref,
                wih_ref, bih_ref, whh_ref, bhh_ref,
                wst_ref, bst_ref, wct_ref, bct_ref,
                wa1_ref, ba1_ref, w2_ref, b2_ref,
                o_ref, h_ref):
    i = pl.program_id(0)

    @pl.when(i == 0)
    def _init():
        h_ref[...] = jnp.zeros_like(h_ref)

    xt = jnp.swapaxes(img_ref[...], 0, 1).reshape(TT * NC, D)
    lens = len_ref[...]
    gi_all = _dot_t(xt, wihl_ref[...]) + bihl_ref[...]
    whhl = whhl_ref[...]
    bhhl = bhhl_ref[...]
    h = h_ref[...]
    for tau in range(TT):
        t = i * TT + tau
        gi = gi_all[tau * NC:(tau + 1) * NC, :]
        gh = _dot_t(h, whhl) + bhhl
        h_new = _gru_gates(gi, gh, h)
        h = jnp.where(t < lens, h_new, h)
    h_ref[...] = h

    @pl.when(i == T // TT - 1)
    def _head():
        cr = h_ref[...]
        gih = _dot_t(cr, wih_ref[...]) + bih_ref[...]
        whh = whh_ref[...]
        bhh = bhh_ref[...]
        row_ids = jax.lax.broadcasted_iota(jnp.int32, (1, NC), 1)

        def step(k, hh):
            onehot = (row_ids == k).astype(jnp.float32)
            g = jnp.dot(onehot, gih, preferred_element_type=jnp.float32)
            gh2 = _dot_t(hh, whh) + bhh
            return _gru_gates(g, gh2, hh)

        h_hi = jnp.zeros((1, H), jnp.float32)
        for k in range(NC):
            h_hi = step(k, h_hi)

        state = jax.nn.relu(_dot_t(h_hi, wst_ref[...]) + bst_ref[...])
        c1024 = jax.nn.relu(_dot_t(cr, wct_ref[...]) + bct_ref[...])
        wa1 = wa1_ref[...]
        s = _dot_t(state, wa1[:, :1024]) + ba1_ref[...]
        P = _dot_t(c1024, wa1[:, 1024:])
        A = P + s
        w2row = w2_ref[...].reshape(1, 1, 1024)

        CH = 8
        chunks = []
        for c in range(NC // CH):
            pc = P[c * CH:(c + 1) * CH, :]
            zq = jnp.maximum(A[:, None, :] + pc[None, :, :], 0.0)
            chunks.append(jnp.sum(zq * w2row, axis=2))
        tab = jnp.concatenate(chunks, axis=1) + b2_ref[...]

        rr = jax.lax.broadcasted_iota(jnp.int32, (NC, NC), 0)
        cc = jax.lax.broadcasted_iota(jnp.int32, (NC, NC), 1)
        valid = rr > cc
        tabm = jnp.where(valid, tab, jnp.float32(-1e30))
        m = jnp.max(tabm)
        e = jnp.where(valid, jnp.exp(tabm - m), 0.0)
        o_ref[...] = e / jnp.sum(e)


@jax.jit
def kernel(images, lengths, Wih_low, Whh_low, bih_low, bhh_low,
           Wih_high, Whh_high, bih_high, bhh_high,
           W_state, b_state, W_cluster, b_cluster,
           W_a1, b_a1, W_a2, b_a2):
    f32 = jnp.float32
    len2 = lengths.astype(jnp.int32).reshape(NC, 1)
    c = lambda shape: pl.BlockSpec(shape, lambda i: tuple(0 for _ in shape))
    probs = pl.pallas_call(
        _fused_body,
        grid=(T // TT,),
        in_specs=[
            c((NC, 1)),
            pl.BlockSpec((NC, TT, D), lambda i: (0, i, 0)),
            c((G3, D)),
            c((1, G3)),
            c((G3, H)),
            c((1, G3)),
            c((G3, H)),
            c((1, G3)),
            c((G3, H)),
            c((1, G3)),
            c((1024, H)),
            c((1, 1024)),
            c((1024, H)),
            c((1, 1024)),
            c((1024, 2048)),
            c((1, 1024)),
            c((1, 1024)),
            c((1, 1)),
        ],
        out_specs=pl.BlockSpec((NC, NC), lambda i: (0, 0)),
        out_shape=jax.ShapeDtypeStruct((NC, NC), f32),
        scratch_shapes=[pltpu.VMEM((NC, H), f32)],
    )(len2, images, Wih_low, bih_low.reshape(1, G3),
      Whh_low, bhh_low.reshape(1, G3),
      Wih_high, bih_high.reshape(1, G3),
      Whh_high, bhh_high.reshape(1, G3),
      W_state, b_state.reshape(1, 1024),
      W_cluster, b_cluster.reshape(1, 1024),
      W_a1, b_a1.reshape(1, 1024),
      W_a2, b_a2.reshape(1, 1))

    row_idx, col_idx = np.tril_indices(NC, k=-1)
    flat_idx = row_idx * NC + col_idx
    idx_pad = np.zeros((2048,), np.int32)
    idx_pad[:2016] = flat_idx
    q_flat = _sc_tril_gather(probs.reshape(NC * NC), jnp.asarray(idx_pad))
    return q_flat[:2016][:, None]


_SC_INFO = plsc.get_sparse_core_info()
_NWORK = _SC_INFO.num_cores * _SC_INFO.num_subcores
_CHUNK = 2048 // _NWORK


@jax.jit
@functools.partial(
    pl.kernel,
    mesh=plsc.VectorSubcoreMesh(core_axis_name="c", subcore_axis_name="s"),
    out_type=jax.ShapeDtypeStruct((2048,), jnp.float32),
    scratch_types=[
        pltpu.VMEM((_CHUNK,), jnp.int32),
        pltpu.VMEM((_CHUNK,), jnp.float32),
        pltpu.SemaphoreType.DMA,
    ],
)
def _sc_tril_gather(table_hbm, idx_hbm, out_hbm, idx_v, vals_v, sem):
    wid = jax.lax.axis_index("s") * _SC_INFO.num_cores + jax.lax.axis_index("c")
    base = wid * _CHUNK
    pltpu.sync_copy(idx_hbm.at[pl.ds(base, _CHUNK)], idx_v)
    pltpu.async_copy(table_hbm.at[idx_v], vals_v, sem).wait()
    pltpu.sync_copy(vals_v, out_hbm.at[pl.ds(base, _CHUNK)])
